# baseline (speedup 1.0000x reference)
import functools

import jax
import jax.numpy as jnp
from jax import lax
from jax.experimental import pallas as pl
from jax.experimental.pallas import tpu as pltpu
from jax.experimental.pallas import tpu_sc as plsc


_LANES = 16


@functools.cache
def _build(batch: int, seq_len: int, d_model: int):
    info = plsc.get_sparse_core_info()
    nc, ns = info.num_cores, info.num_subcores
    nw = nc * ns
    pos_per_w = seq_len // nw
    chunk = 8
    rows_ring = 8
    stage_ring = 4
    n_qq = pos_per_w // chunk
    n_chunks = n_qq * batch
    mesh = plsc.VectorSubcoreMesh(core_axis_name="c", subcore_axis_name="s")

    @functools.partial(
        pl.kernel,
        out_type=jax.ShapeDtypeStruct((batch, seq_len, d_model), jnp.float32),
        mesh=mesh,
        scratch_types=[
            pltpu.VMEM((batch, pos_per_w), jnp.int32),
            pltpu.VMEM((rows_ring * chunk, d_model), jnp.float32),
            pltpu.VMEM((2 * chunk, d_model), jnp.float32),
            pltpu.VMEM((stage_ring * chunk, d_model), jnp.float32),
            pltpu.SemaphoreType.DMA,
            pltpu.SemaphoreType.DMA,
            pltpu.SemaphoreType.DMA,
        ],
    )
    def emb(x_hbm, tok_hbm, pos_hbm, out_hbm, idx_v, rows_v, pos_v, stage_v,
            gsem, psem, ssem):
        wid = lax.axis_index("s") * nc + lax.axis_index("c")
        s0 = wid * pos_per_w
        for bb in range(batch):
            pltpu.sync_copy(x_hbm.at[bb, pl.ds(s0, pos_per_w)], idx_v.at[bb])

        def gather_desc(t):
            qq, bb = t // batch, t % batch
            slot = (t % rows_ring) * chunk
            return pltpu.make_async_copy(
                tok_hbm.at[idx_v.at[bb, pl.ds(qq * chunk, chunk)]],
                rows_v.at[pl.ds(slot, chunk)],
                gsem,
            )

        def pos_desc(qq):
            half = (qq % 2) * chunk
            return pltpu.make_async_copy(
                pos_hbm.at[pl.ds(s0 + qq * chunk, chunk)],
                pos_v.at[pl.ds(half, chunk)],
                psem,
            )

        def store_desc(t):
            qq, bb = t // batch, t % batch
            half = (t % stage_ring) * chunk
            return pltpu.make_async_copy(
                stage_v.at[pl.ds(half, chunk)],
                out_hbm.at[bb, pl.ds(s0 + qq * chunk, chunk)],
                ssem,
            )

        for t0 in range(rows_ring):
            gather_desc(t0).start()
        pos_desc(0).start()

        @pl.loop(0, n_chunks)
        def _(t):
            qq = t // batch
            bb = lax.rem(t, batch)
            gslot = lax.rem(t, rows_ring) * chunk
            half = lax.rem(t, stage_ring) * chunk
            phalf = lax.rem(qq, 2) * chunk

            @pl.when(bb == 0)
            def _():
                pos_desc(qq).wait()

            gather_desc(t).wait()

            @pl.when(t >= stage_ring)
            def _():
                store_desc(t - stage_ring).wait()

            @plsc.parallel_loop(0, chunk * d_model, _LANES, unroll=8)
            def _(i):
                r = i // d_model
                col = i % d_model
                stage_v[half + r, pl.ds(col, _LANES)] = (
                    rows_v[gslot + r, pl.ds(col, _LANES)]
                    + pos_v[phalf + r, pl.ds(col, _LANES)]
                )

            store_desc(t).start()

            @pl.when(t + rows_ring < n_chunks)
            def _():
                gather_desc(t + rows_ring).start()

            @pl.when(jnp.logical_and(bb == 1, qq + 1 < n_qq))
            def _():
                pos_desc(qq + 1).start()

        for t0 in range(n_chunks - stage_ring, n_chunks):
            store_desc(t0).wait()

    return emb


def kernel(x, token_table, pos_table):
    batch, seq_len = x.shape
    d_model = token_table.shape[1]
    emb = _build(batch, seq_len, d_model)
    return emb(x.astype(jnp.int32), token_table, pos_table)


# speedup vs baseline: 1.0092x; 1.0092x over previous
import functools

import jax
import jax.numpy as jnp
from jax import lax
from jax.experimental import pallas as pl
from jax.experimental.pallas import tpu as pltpu
from jax.experimental.pallas import tpu_sc as plsc


_LANES = 16


@functools.cache
def _build(batch: int, seq_len: int, d_model: int):
    info = plsc.get_sparse_core_info()
    nc, ns = info.num_cores, info.num_subcores
    nw = nc * ns
    pos_per_w = seq_len // nw
    chunk = 8
    rows_ring = 6
    stage_ring = 6
    n_qq = pos_per_w // chunk
    n_chunks = n_qq * batch
    mesh = plsc.VectorSubcoreMesh(core_axis_name="c", subcore_axis_name="s")

    @functools.partial(
        pl.kernel,
        out_type=jax.ShapeDtypeStruct((batch, seq_len, d_model), jnp.float32),
        mesh=mesh,
        scratch_types=[
            pltpu.VMEM((batch, pos_per_w), jnp.int32),
            pltpu.VMEM((rows_ring * chunk, d_model), jnp.float32),
            pltpu.VMEM((2 * chunk, d_model), jnp.float32),
            pltpu.VMEM((stage_ring * chunk, d_model), jnp.float32),
            pltpu.SemaphoreType.DMA,
            pltpu.SemaphoreType.DMA,
            pltpu.SemaphoreType.DMA,
        ],
    )
    def emb(x_hbm, tok_hbm, pos_hbm, out_hbm, idx_v, rows_v, pos_v, stage_v,
            gsem, psem, ssem):
        wid = lax.axis_index("s") * nc + lax.axis_index("c")
        s0 = wid * pos_per_w
        for bb in range(batch):
            pltpu.sync_copy(x_hbm.at[bb, pl.ds(s0, pos_per_w)], idx_v.at[bb])

        def gather_desc(t):
            qq, bb = t // batch, t % batch
            slot = (t % rows_ring) * chunk
            return pltpu.make_async_copy(
                tok_hbm.at[idx_v.at[bb, pl.ds(qq * chunk, chunk)]],
                rows_v.at[pl.ds(slot, chunk)],
                gsem,
            )

        def pos_desc(qq):
            half = (qq % 2) * chunk
            return pltpu.make_async_copy(
                pos_hbm.at[pl.ds(s0 + qq * chunk, chunk)],
                pos_v.at[pl.ds(half, chunk)],
                psem,
            )

        def store_desc(t):
            qq, bb = t // batch, t % batch
            half = (t % stage_ring) * chunk
            return pltpu.make_async_copy(
                stage_v.at[pl.ds(half, chunk)],
                out_hbm.at[bb, pl.ds(s0 + qq * chunk, chunk)],
                ssem,
            )

        for t0 in range(rows_ring):
            gather_desc(t0).start()
        pos_desc(0).start()

        @pl.loop(0, n_chunks)
        def _(t):
            qq = t // batch
            bb = lax.rem(t, batch)
            gslot = lax.rem(t, rows_ring) * chunk
            half = lax.rem(t, stage_ring) * chunk
            phalf = lax.rem(qq, 2) * chunk

            @pl.when(bb == 0)
            def _():
                pos_desc(qq).wait()

            gather_desc(t).wait()

            @pl.when(t >= stage_ring)
            def _():
                store_desc(t - stage_ring).wait()

            @plsc.parallel_loop(0, chunk * d_model, _LANES, unroll=8)
            def _(i):
                r = i // d_model
                col = i % d_model
                stage_v[half + r, pl.ds(col, _LANES)] = (
                    rows_v[gslot + r, pl.ds(col, _LANES)]
                    + pos_v[phalf + r, pl.ds(col, _LANES)]
                )

            store_desc(t).start()

            @pl.when(t + rows_ring < n_chunks)
            def _():
                gather_desc(t + rows_ring).start()

            @pl.when(jnp.logical_and(bb == 1, qq + 1 < n_qq))
            def _():
                pos_desc(qq + 1).start()

        for t0 in range(n_chunks - stage_ring, n_chunks):
            store_desc(t0).wait()

    return emb


def kernel(x, token_table, pos_table):
    batch, seq_len = x.shape
    d_model = token_table.shape[1]
    emb = _build(batch, seq_len, d_model)
    return emb(x.astype(jnp.int32), token_table, pos_table)


# speedup vs baseline: 1.0276x; 1.0182x over previous
import functools

import jax
import jax.numpy as jnp
from jax import lax
from jax.experimental import pallas as pl
from jax.experimental.pallas import tpu as pltpu
from jax.experimental.pallas import tpu_sc as plsc


_LANES = 16


@functools.cache
def _build(batch: int, seq_len: int, d_model: int):
    info = plsc.get_sparse_core_info()
    nc, ns = info.num_cores, info.num_subcores
    nw = nc * ns
    pos_per_w = seq_len // nw
    chunk = 8
    rows_ring = 6
    stage_ring = 6
    n_qq = pos_per_w // chunk
    n_chunks = n_qq * batch
    mesh = plsc.VectorSubcoreMesh(core_axis_name="c", subcore_axis_name="s")

    @functools.partial(
        pl.kernel,
        out_type=jax.ShapeDtypeStruct((batch, seq_len, d_model), jnp.float32),
        mesh=mesh,
        scratch_types=[
            pltpu.VMEM((batch, pos_per_w), jnp.int32),
            pltpu.VMEM((rows_ring * chunk, d_model), jnp.float32),
            pltpu.VMEM((2 * chunk, d_model), jnp.float32),
            pltpu.VMEM((stage_ring * chunk, d_model), jnp.float32),
            pltpu.SemaphoreType.DMA,
            pltpu.SemaphoreType.DMA,
            pltpu.SemaphoreType.DMA,
        ],
    )
    def emb(x_hbm, tok_hbm, pos_hbm, out_hbm, idx_v, rows_v, pos_v, stage_v,
            gsem, psem, ssem):
        wid = lax.axis_index("s") * nc + lax.axis_index("c")
        s0 = wid * pos_per_w
        idx_copies = [
            pltpu.async_copy(x_hbm.at[bb, pl.ds(s0, pos_per_w)], idx_v.at[bb], ssem)
            for bb in range(batch)
        ]
        for cp in idx_copies:
            cp.wait()

        def gather_desc(t):
            qq, bb = t // batch, t % batch
            slot = (t % rows_ring) * chunk
            return pltpu.make_async_copy(
                tok_hbm.at[idx_v.at[bb, pl.ds(qq * chunk, chunk)]],
                rows_v.at[pl.ds(slot, chunk)],
                gsem,
            )

        def pos_desc(qq):
            half = (qq % 2) * chunk
            return pltpu.make_async_copy(
                pos_hbm.at[pl.ds(s0 + qq * chunk, chunk)],
                pos_v.at[pl.ds(half, chunk)],
                psem,
            )

        def store_desc(t):
            qq, bb = t // batch, t % batch
            half = (t % stage_ring) * chunk
            return pltpu.make_async_copy(
                stage_v.at[pl.ds(half, chunk)],
                out_hbm.at[bb, pl.ds(s0 + qq * chunk, chunk)],
                ssem,
            )

        for t0 in range(rows_ring):
            gather_desc(t0).start()
        pos_desc(0).start()

        @pl.loop(0, n_chunks)
        def _(t):
            qq = t // batch
            bb = lax.rem(t, batch)
            gslot = lax.rem(t, rows_ring) * chunk
            half = lax.rem(t, stage_ring) * chunk
            phalf = lax.rem(qq, 2) * chunk

            @pl.when(bb == 0)
            def _():
                pos_desc(qq).wait()

            gather_desc(t).wait()

            @pl.when(t >= stage_ring)
            def _():
                store_desc(t - stage_ring).wait()

            @plsc.parallel_loop(0, chunk * d_model, _LANES, unroll=8)
            def _(i):
                r = i // d_model
                col = i % d_model
                stage_v[half + r, pl.ds(col, _LANES)] = (
                    rows_v[gslot + r, pl.ds(col, _LANES)]
                    + pos_v[phalf + r, pl.ds(col, _LANES)]
                )

            store_desc(t).start()

            @pl.when(t + rows_ring < n_chunks)
            def _():
                gather_desc(t + rows_ring).start()

            @pl.when(jnp.logical_and(bb == 1, qq + 1 < n_qq))
            def _():
                pos_desc(qq + 1).start()

        for t0 in range(n_chunks - stage_ring, n_chunks):
            store_desc(t0).wait()

    return emb


def kernel(x, token_table, pos_table):
    batch, seq_len = x.shape
    d_model = token_table.shape[1]
    emb = _build(batch, seq_len, d_model)
    return emb(x.astype(jnp.int32), token_table, pos_table)
